# baseline (speedup 1.0000x reference)
import functools

import jax
import jax.numpy as jnp
from jax import lax
from jax.experimental import pallas as pl
from jax.experimental.pallas import tpu as pltpu
from jax.experimental.pallas import tpu_sc as plsc

N = 10000
NP = 10240
D_IN = 128
DH = 128
DO = 64
E = 320000
NC, NS = 2, 16
NT = NC * NS
CH = 128
NCHUNK = 80
NCHUNK2 = 160
EPT = CH * NCHUNK
EP = EPT * NT
RPS = NP // NS
DEGW = 16

_mesh = functools.partial(
    plsc.VectorSubcoreMesh,
    core_axis_name="c", subcore_axis_name="s", num_cores=NC, num_subcores=NS,
)

_sc_params = pltpu.CompilerParams(use_tc_tiling_on_sc=False)


def _zero_fill(buf, f):
    @pl.loop(0, 64)
    def _(i):
        for k in range(f // 16):
            buf[i, pl.ds(16 * k, 16)] = jnp.zeros((16,), jnp.float32)


def _zero_shared(acc_sh, zbuf, sid):
    @pl.loop(0, RPS // 64)
    def _(j):
        pltpu.sync_copy(zbuf, acc_sh.at[pl.ds(sid * RPS + j * 64, 64)])


_RING = 5


@functools.partial(
    pl.kernel,
    out_type=jax.ShapeDtypeStruct((NC, NP, DEGW), jnp.float32),
    mesh=_mesh(),
    compiler_params=_sc_params,
    scratch_types=[
        [pltpu.VMEM((2, CH), jnp.int32) for _ in range(_RING)],
        pltpu.VMEM((CH, DEGW), jnp.float32),
        pltpu.VMEM((64, DEGW), jnp.float32),
        pltpu.VMEM_SHARED((NP, DEGW), jnp.float32),
        [pltpu.SemaphoreType.DMA for _ in range(_RING)],
    ],
)
def _deg_pass(e_hbm, out_hbm, idx, ones_v, zbuf, acc_sh, ssem):
    cid = lax.axis_index("c")
    sid = lax.axis_index("s")
    wid = cid * NS + sid

    @pl.loop(0, CH)
    def _(i):
        ones_v[i, :] = jnp.ones((16,), jnp.float32)

    _zero_fill(zbuf, DEGW)
    _zero_shared(acc_sh, zbuf, sid)
    plsc.subcore_barrier()

    base = wid * NCHUNK

    def wait_scatter(r):
        pltpu.make_async_copy(ones_v, acc_sh.at[pl.ds(0, CH)], ssem[r]).wait()

    @pl.loop(0, NCHUNK // _RING)
    def _(p):
        g0 = base + _RING * p
        for r in range(_RING):
            @pl.when(p > 0)
            def _(r=r):
                wait_scatter(r)
            pltpu.sync_copy(e_hbm.at[:, pl.ds((g0 + r) * CH, CH)], idx[r])
            pltpu.async_copy(ones_v, acc_sh.at[idx[r].at[1]], ssem[r],
                             add=True)

    for r in range(_RING):
        wait_scatter(r)
    plsc.subcore_barrier()
    pltpu.sync_copy(acc_sh.at[pl.ds(sid * RPS, RPS)],
                    out_hbm.at[cid, pl.ds(sid * RPS, RPS)])


def _make_edge_pass(f, ring):
    h = f // 2
    ngroup = NCHUNK2 // ring

    @functools.partial(
        pl.kernel,
        out_type=jax.ShapeDtypeStruct((NC, NP, h), jnp.float32),
        mesh=_mesh(),
        compiler_params=_sc_params,
        scratch_types=[
            [pltpu.VMEM((2, CH), jnp.int32) for _ in range(ring)],
            [pltpu.VMEM((CH, h), jnp.float32) for _ in range(ring)],
            pltpu.VMEM((64, h), jnp.float32),
            pltpu.VMEM_SHARED((NP, h), jnp.float32),
            pltpu.VMEM_SHARED((NP, h), jnp.float32),
            [pltpu.SemaphoreType.DMA for _ in range(ring)],
            [pltpu.SemaphoreType.DMA for _ in range(ring)],
        ],
    )
    def edge_pass(hs_hbm, e_hbm, out_hbm,
                  idx, rows, zbuf, tab_sh, acc_sh, gsem, ssem):
        cid = lax.axis_index("c")
        sid = lax.axis_index("s")

        pltpu.sync_copy(hs_hbm.at[cid, pl.ds(sid * RPS, RPS)],
                        tab_sh.at[pl.ds(sid * RPS, RPS)])
        _zero_fill(zbuf, h)
        _zero_shared(acc_sh, zbuf, sid)
        plsc.subcore_barrier()

        base = sid * NCHUNK2

        def wait_gather(r):
            pltpu.make_async_copy(hs_hbm.at[0, pl.ds(0, CH)],
                                  rows[r], gsem[r]).wait()

        def wait_scatter(r):
            pltpu.make_async_copy(rows[r], acc_sh.at[pl.ds(0, CH)],
                                  ssem[r]).wait()

        @pl.loop(0, ngroup)
        def _(p):
            g0 = base + ring * p
            for r in range(ring):
                @pl.when(p > 0)
                def _(r=r):
                    wait_scatter(r)
                pltpu.sync_copy(e_hbm.at[0, pl.ds((g0 + r) * CH, CH)],
                                idx[r].at[0])
                pltpu.sync_copy(e_hbm.at[1, pl.ds((g0 + r) * CH, CH)],
                                idx[r].at[1])
                pltpu.async_copy(tab_sh.at[idx[r].at[0]], rows[r], gsem[r])
            for r in range(ring):
                wait_gather(r)
                pltpu.async_copy(rows[r], acc_sh.at[idx[r].at[1]],
                                 ssem[r], add=True)

        for r in range(ring):
            wait_scatter(r)
        plsc.subcore_barrier()
        pltpu.sync_copy(acc_sh.at[pl.ds(sid * RPS, RPS)],
                        out_hbm.at[cid, pl.ds(sid * RPS, RPS)])

    return edge_pass


_edge_pass_h = _make_edge_pass(DH, 5)
_edge_pass_o = _make_edge_pass(DO, 8)


def _dis_from_deg(deg_ref):
    deg = deg_ref[0, :, 0:1] + deg_ref[1, :, 0:1]
    rows = lax.broadcasted_iota(jnp.int32, (NP, 1), 0)
    deg = deg + jnp.where(rows < N, 1.0, 0.0)
    return jnp.where(deg > 0, lax.rsqrt(deg), 0.0)


def _tc1_body(x_ref, w_ref, deg_ref, o_ref):
    dis = _dis_from_deg(deg_ref)
    hh = jnp.dot(x_ref[...], w_ref[...],
                 preferred_element_type=jnp.float32) * dis[:N]
    zt = jnp.zeros((NP - N, DH // 2), jnp.float32)
    o_ref[0] = jnp.concatenate([hh[:, :DH // 2], zt], axis=0)
    o_ref[1] = jnp.concatenate([hh[:, DH // 2:], zt], axis=0)


def _tc2_body(acc_ref, hs1_ref, deg_ref, b1_ref, g_ref, be_ref, w_ref, o_ref):
    dis = _dis_from_deg(deg_ref)
    acc = jnp.concatenate([acc_ref[0] + hs1_ref[0],
                           acc_ref[1] + hs1_ref[1]], axis=1)
    t = acc * dis + b1_ref[...]
    t = jnp.maximum(t, 0.0)
    mu = jnp.mean(t, axis=-1, keepdims=True)
    var = jnp.mean((t - mu) ** 2, axis=-1, keepdims=True)
    h = (t - mu) * lax.rsqrt(var + 1e-5) * g_ref[...] + be_ref[...]
    hs2 = jnp.dot(h, w_ref[...], preferred_element_type=jnp.float32) * dis
    o_ref[0] = hs2[:, :DO // 2]
    o_ref[1] = hs2[:, DO // 2:]


def _tc3_body(acc_ref, hs2_ref, deg_ref, b2_ref, o_ref):
    dis = _dis_from_deg(deg_ref)
    res = jnp.concatenate([acc_ref[0] + hs2_ref[0],
                           acc_ref[1] + hs2_ref[1]], axis=1)
    o_ref[...] = (res * dis + b2_ref[...])[:N]


_tc1 = pl.pallas_call(_tc1_body,
                      out_shape=jax.ShapeDtypeStruct((NC, NP, DH // 2),
                                                     jnp.float32))
_tc2 = pl.pallas_call(_tc2_body,
                      out_shape=jax.ShapeDtypeStruct((NC, NP, DO // 2),
                                                     jnp.float32))
_tc3 = pl.pallas_call(_tc3_body,
                      out_shape=jax.ShapeDtypeStruct((N, DO), jnp.float32))


def kernel(x, edge_index, W1, b1, gamma, beta, W2, b2):
    ei = edge_index.astype(jnp.int32)
    pad = jnp.full((2, EP - E), N, jnp.int32)
    e3 = jnp.concatenate([ei, pad], axis=1)

    degp = _deg_pass(e3)
    hs1 = _tc1(x, W1, degp)
    acc1 = _edge_pass_h(hs1, e3)
    hs2 = _tc2(acc1, hs1, degp, b1.reshape(1, DH),
               gamma.reshape(1, DH), beta.reshape(1, DH), W2)
    acc2 = _edge_pass_o(hs2, e3)
    return _tc3(acc2, hs2, degp, b2.reshape(1, DO))


# speedup vs baseline: 1.2535x; 1.2535x over previous
import functools

import jax
import jax.numpy as jnp
from jax import lax
from jax.experimental import pallas as pl
from jax.experimental.pallas import tpu as pltpu
from jax.experimental.pallas import tpu_sc as plsc

N = 10000
NP = 10240
D_IN = 128
DH = 128
DO = 64
E = 320000
NC, NS = 2, 16
NT = NC * NS
CH = 128
NCHUNK = 80
NCHUNK2 = 160
EPT = CH * NCHUNK
EP = EPT * NT
RPS = NP // NS
DEGW = 16

_mesh = functools.partial(
    plsc.VectorSubcoreMesh,
    core_axis_name="c", subcore_axis_name="s", num_cores=NC, num_subcores=NS,
)

_sc_params = pltpu.CompilerParams(use_tc_tiling_on_sc=False)


def _zero_fill(buf, f):
    @pl.loop(0, 64)
    def _(i):
        for k in range(f // 16):
            buf[i, pl.ds(16 * k, 16)] = jnp.zeros((16,), jnp.float32)


def _zero_shared(acc_sh, zbuf, sid):
    @pl.loop(0, RPS // 64)
    def _(j):
        pltpu.sync_copy(zbuf, acc_sh.at[pl.ds(sid * RPS + j * 64, 64)])


_RING = 5


@functools.partial(
    pl.kernel,
    out_type=jax.ShapeDtypeStruct((NC, NP, DEGW), jnp.float32),
    mesh=_mesh(),
    compiler_params=_sc_params,
    scratch_types=[
        [pltpu.VMEM((2, CH), jnp.int32) for _ in range(_RING)],
        pltpu.VMEM((CH, DEGW), jnp.float32),
        pltpu.VMEM((64, DEGW), jnp.float32),
        pltpu.VMEM_SHARED((NP, DEGW), jnp.float32),
        [pltpu.SemaphoreType.DMA for _ in range(_RING)],
    ],
)
def _deg_pass(e_hbm, out_hbm, idx, ones_v, zbuf, acc_sh, ssem):
    cid = lax.axis_index("c")
    sid = lax.axis_index("s")
    wid = cid * NS + sid

    @pl.loop(0, CH)
    def _(i):
        ones_v[i, :] = jnp.ones((16,), jnp.float32)

    _zero_fill(zbuf, DEGW)
    _zero_shared(acc_sh, zbuf, sid)
    plsc.subcore_barrier()

    base = wid * NCHUNK

    def wait_scatter(r):
        pltpu.make_async_copy(ones_v, acc_sh.at[pl.ds(0, CH)], ssem[r]).wait()

    @pl.loop(0, NCHUNK // _RING)
    def _(p):
        g0 = base + _RING * p
        for r in range(_RING):
            @pl.when(p > 0)
            def _(r=r):
                wait_scatter(r)
            pltpu.sync_copy(e_hbm.at[g0 + r], idx[r])
            pltpu.async_copy(ones_v, acc_sh.at[idx[r].at[1]], ssem[r],
                             add=True)

    for r in range(_RING):
        wait_scatter(r)
    plsc.subcore_barrier()
    pltpu.sync_copy(acc_sh.at[pl.ds(sid * RPS, RPS)],
                    out_hbm.at[cid, pl.ds(sid * RPS, RPS)])


def _make_edge_pass(f, ring):
    h = f // 2
    ngroup = NCHUNK2 // ring

    @functools.partial(
        pl.kernel,
        out_type=jax.ShapeDtypeStruct((NC, NP, h), jnp.float32),
        mesh=_mesh(),
        compiler_params=_sc_params,
        scratch_types=[
            [pltpu.VMEM((2, CH), jnp.int32) for _ in range(ring)],
            [pltpu.VMEM((CH, h), jnp.float32) for _ in range(ring)],
            pltpu.VMEM((64, h), jnp.float32),
            pltpu.VMEM_SHARED((NP, h), jnp.float32),
            pltpu.VMEM_SHARED((NP, h), jnp.float32),
            [pltpu.SemaphoreType.DMA for _ in range(ring)],
            [pltpu.SemaphoreType.DMA for _ in range(ring)],
        ],
    )
    def edge_pass(hs_hbm, e_hbm, out_hbm,
                  idx, rows, zbuf, tab_sh, acc_sh, gsem, ssem):
        cid = lax.axis_index("c")
        sid = lax.axis_index("s")

        pltpu.sync_copy(hs_hbm.at[cid, pl.ds(sid * RPS, RPS)],
                        tab_sh.at[pl.ds(sid * RPS, RPS)])
        _zero_fill(zbuf, h)
        _zero_shared(acc_sh, zbuf, sid)
        plsc.subcore_barrier()

        base = sid * NCHUNK2

        def wait_gather(r):
            pltpu.make_async_copy(hs_hbm.at[0, pl.ds(0, CH)],
                                  rows[r], gsem[r]).wait()

        def wait_scatter(r):
            pltpu.make_async_copy(rows[r], acc_sh.at[pl.ds(0, CH)],
                                  ssem[r]).wait()

        @pl.loop(0, ngroup)
        def _(p):
            g0 = base + ring * p
            for r in range(ring):
                @pl.when(p > 0)
                def _(r=r):
                    wait_scatter(r)
                pltpu.sync_copy(e_hbm.at[g0 + r], idx[r])
                pltpu.async_copy(tab_sh.at[idx[r].at[0]], rows[r], gsem[r])
            for r in range(ring):
                wait_gather(r)
                pltpu.async_copy(rows[r], acc_sh.at[idx[r].at[1]],
                                 ssem[r], add=True)

        for r in range(ring):
            wait_scatter(r)
        plsc.subcore_barrier()
        pltpu.sync_copy(acc_sh.at[pl.ds(sid * RPS, RPS)],
                        out_hbm.at[cid, pl.ds(sid * RPS, RPS)])

    return edge_pass


_edge_pass_h = _make_edge_pass(DH, 5)
_edge_pass_o = _make_edge_pass(DO, 8)


def _dis_from_deg(deg_ref):
    deg = deg_ref[0, :, 0:1] + deg_ref[1, :, 0:1]
    rows = lax.broadcasted_iota(jnp.int32, (NP, 1), 0)
    deg = deg + jnp.where(rows < N, 1.0, 0.0)
    return jnp.where(deg > 0, lax.rsqrt(deg), 0.0)


def _tc1_body(x_ref, w_ref, deg_ref, o_ref):
    dis = _dis_from_deg(deg_ref)
    hh = jnp.dot(x_ref[...], w_ref[...],
                 preferred_element_type=jnp.float32) * dis[:N]
    zt = jnp.zeros((NP - N, DH // 2), jnp.float32)
    o_ref[0] = jnp.concatenate([hh[:, :DH // 2], zt], axis=0)
    o_ref[1] = jnp.concatenate([hh[:, DH // 2:], zt], axis=0)


def _tc2_body(acc_ref, hs1_ref, deg_ref, b1_ref, g_ref, be_ref, w_ref, o_ref):
    dis = _dis_from_deg(deg_ref)
    acc = jnp.concatenate([acc_ref[0] + hs1_ref[0],
                           acc_ref[1] + hs1_ref[1]], axis=1)
    t = acc * dis + b1_ref[...]
    t = jnp.maximum(t, 0.0)
    mu = jnp.mean(t, axis=-1, keepdims=True)
    var = jnp.mean((t - mu) ** 2, axis=-1, keepdims=True)
    h = (t - mu) * lax.rsqrt(var + 1e-5) * g_ref[...] + be_ref[...]
    hs2 = jnp.dot(h, w_ref[...], preferred_element_type=jnp.float32) * dis
    o_ref[0] = hs2[:, :DO // 2]
    o_ref[1] = hs2[:, DO // 2:]


def _tc3_body(acc_ref, hs2_ref, deg_ref, b2_ref, o_ref):
    dis = _dis_from_deg(deg_ref)
    res = jnp.concatenate([acc_ref[0] + hs2_ref[0],
                           acc_ref[1] + hs2_ref[1]], axis=1)
    o_ref[...] = (res * dis + b2_ref[...])[:N]


_tc1 = pl.pallas_call(_tc1_body,
                      out_shape=jax.ShapeDtypeStruct((NC, NP, DH // 2),
                                                     jnp.float32))
_tc2 = pl.pallas_call(_tc2_body,
                      out_shape=jax.ShapeDtypeStruct((NC, NP, DO // 2),
                                                     jnp.float32))
_tc3 = pl.pallas_call(_tc3_body,
                      out_shape=jax.ShapeDtypeStruct((N, DO), jnp.float32))


def kernel(x, edge_index, W1, b1, gamma, beta, W2, b2):
    ei = edge_index.astype(jnp.int32)
    pad = jnp.full((2, EP - E), N, jnp.int32)
    e3 = jnp.concatenate([ei, pad], axis=1)
    e3 = e3.reshape(2, NT * NCHUNK, CH).transpose(1, 0, 2)

    degp = _deg_pass(e3)
    hs1 = _tc1(x, W1, degp)
    acc1 = _edge_pass_h(hs1, e3)
    hs2 = _tc2(acc1, hs1, degp, b1.reshape(1, DH),
               gamma.reshape(1, DH), beta.reshape(1, DH), W2)
    acc2 = _edge_pass_o(hs2, e3)
    return _tc3(acc2, hs2, degp, b2.reshape(1, DO))


# speedup vs baseline: 1.3407x; 1.0695x over previous
import functools

import jax
import jax.numpy as jnp
from jax import lax
from jax.experimental import pallas as pl
from jax.experimental.pallas import tpu as pltpu
from jax.experimental.pallas import tpu_sc as plsc

N = 10000
NP = 10240
D_IN = 128
DH = 128
DO = 64
E = 320000
NC, NS = 2, 16
NT = NC * NS
CH = 128
NCHUNK = 80
NCHUNK2 = 160
EPT = CH * NCHUNK
EP = EPT * NT
RPS = NP // NS
DEGW = 8

_mesh = functools.partial(
    plsc.VectorSubcoreMesh,
    core_axis_name="c", subcore_axis_name="s", num_cores=NC, num_subcores=NS,
)

_sc_params = pltpu.CompilerParams(use_tc_tiling_on_sc=False)


def _zero_fill(buf, nrows, f):
    @pl.loop(0, nrows)
    def _(i):
        for k in range(f // 16):
            buf[i, pl.ds(16 * k, 16)] = jnp.zeros((16,), jnp.float32)


def _zero_shared(acc_sh, zbuf, nrows, sid):
    @pl.loop(0, RPS // nrows)
    def _(j):
        pltpu.sync_copy(zbuf, acc_sh.at[pl.ds(sid * RPS + j * nrows, nrows)])


_RING = 5


@functools.partial(
    pl.kernel,
    out_type=jax.ShapeDtypeStruct((NC, NP, DEGW), jnp.float32),
    mesh=_mesh(),
    compiler_params=_sc_params,
    scratch_types=[
        pltpu.VMEM((NCHUNK, 2, CH), jnp.int32),
        pltpu.VMEM((64 + CH, DEGW), jnp.float32),
        pltpu.VMEM_SHARED((NP, DEGW), jnp.float32),
        [pltpu.SemaphoreType.DMA for _ in range(_RING)],
        pltpu.SemaphoreType.DMA,
    ],
)
def _deg_pass(e_hbm, oz_hbm, out_hbm, idx_all, ozv, acc_sh, ssem, esem):
    cid = lax.axis_index("c")
    sid = lax.axis_index("s")
    wid = cid * NS + sid

    eload = pltpu.async_copy(e_hbm.at[pl.ds(wid * NCHUNK, NCHUNK)],
                             idx_all, esem)
    pltpu.sync_copy(oz_hbm, ozv)
    ones_v = ozv.at[pl.ds(64, CH)]
    _zero_shared(acc_sh, ozv.at[pl.ds(0, 64)], 64, sid)
    eload.wait()
    plsc.subcore_barrier()

    def wait_scatter(r):
        pltpu.make_async_copy(ones_v, acc_sh.at[pl.ds(0, CH)], ssem[r]).wait()

    @pl.loop(0, NCHUNK // _RING)
    def _(p):
        g0 = _RING * p
        for r in range(_RING):
            @pl.when(p > 0)
            def _(r=r):
                wait_scatter(r)
            pltpu.async_copy(ones_v, acc_sh.at[idx_all.at[g0 + r, 1]],
                             ssem[r], add=True)

    for r in range(_RING):
        wait_scatter(r)
    plsc.subcore_barrier()
    pltpu.sync_copy(acc_sh.at[pl.ds(sid * RPS, RPS)],
                    out_hbm.at[cid, pl.ds(sid * RPS, RPS)])


def _make_edge_pass(f, ring, blk, finale=False):
    h = f // 2
    nblk = NCHUNK2 // blk
    ngroup = blk // ring

    extra_in = ([jax.ShapeDtypeStruct((NP, h), jnp.float32),
                 jax.ShapeDtypeStruct((NC, 1, h), jnp.float32)]
                if finale else [])

    @functools.partial(
        pl.kernel,
        out_type=jax.ShapeDtypeStruct((NC, NP, h), jnp.float32),
        mesh=_mesh(),
        compiler_params=_sc_params,
        scratch_types=[
            pltpu.VMEM((blk, 2, CH), jnp.int32),
            [pltpu.VMEM((CH, h), jnp.float32) for _ in range(ring)],
            pltpu.VMEM((1, h), jnp.float32),
            pltpu.VMEM_SHARED((NP, h), jnp.float32),
            pltpu.VMEM_SHARED((NP, h), jnp.float32),
            [pltpu.SemaphoreType.DMA for _ in range(ring)],
            [pltpu.SemaphoreType.DMA for _ in range(ring)],
            pltpu.SemaphoreType.DMA,
        ],
    )
    def edge_pass(hs_hbm, e_hbm, *rest):
        if finale:
            (disb_hbm, bb_hbm, out_hbm,
             idx_all, rows, bbv, tab_sh, acc_sh, gsem, ssem, esem) = rest
        else:
            (out_hbm,
             idx_all, rows, bbv, tab_sh, acc_sh, gsem, ssem, esem) = rest
        cid = lax.axis_index("c")
        sid = lax.axis_index("s")

        eload = pltpu.async_copy(e_hbm.at[pl.ds(sid * NCHUNK2, blk)],
                                 idx_all, esem)
        pltpu.sync_copy(hs_hbm.at[cid, pl.ds(sid * RPS, RPS)],
                        tab_sh.at[pl.ds(sid * RPS, RPS)])
        _zero_fill(rows[0], CH, h)
        _zero_shared(acc_sh, rows[0], CH, sid)
        eload.wait()
        plsc.subcore_barrier()

        def wait_gather(r):
            pltpu.make_async_copy(hs_hbm.at[0, pl.ds(0, CH)],
                                  rows[r], gsem[r]).wait()

        def wait_scatter(r):
            pltpu.make_async_copy(rows[r], acc_sh.at[pl.ds(0, CH)],
                                  ssem[r]).wait()

        @pl.loop(0, nblk)
        def _(b):
            @pl.when(b > 0)
            def _():
                for r in range(ring):
                    wait_scatter(r)
                pltpu.sync_copy(
                    e_hbm.at[pl.ds(sid * NCHUNK2 + b * blk, blk)], idx_all)

            @pl.loop(0, ngroup)
            def _(p):
                g0 = ring * p
                for r in range(ring):
                    @pl.when(p > 0)
                    def _(r=r):
                        wait_scatter(r)
                    pltpu.async_copy(tab_sh.at[idx_all.at[g0 + r, 0]],
                                     rows[r], gsem[r])
                for r in range(ring):
                    wait_gather(r)
                    pltpu.async_copy(rows[r], acc_sh.at[idx_all.at[g0 + r, 1]],
                                     ssem[r], add=True)

        for r in range(ring):
            wait_scatter(r)
        plsc.subcore_barrier()
        if not finale:
            pltpu.sync_copy(acc_sh.at[pl.ds(sid * RPS, RPS)],
                            out_hbm.at[cid, pl.ds(sid * RPS, RPS)])
        else:
            pltpu.sync_copy(bb_hbm.at[cid], bbv)
            for blk_i in range(RPS // CH):
                rbase = sid * RPS + blk_i * CH
                pltpu.sync_copy(acc_sh.at[pl.ds(rbase, CH)], rows[0])
                pltpu.sync_copy(tab_sh.at[pl.ds(rbase, CH)], rows[1])
                pltpu.sync_copy(disb_hbm.at[pl.ds(rbase, CH)], rows[2])

                @pl.loop(0, CH)
                def _(i):
                    for k in range(h // 16):
                        sl = pl.ds(16 * k, 16)
                        rows[0][i, sl] = ((rows[0][i, sl] + rows[1][i, sl])
                                          * rows[2][i, sl] + bbv[0, sl])
                pltpu.sync_copy(rows[0], out_hbm.at[cid, pl.ds(rbase, CH)])

    return edge_pass


_edge_pass_h = _make_edge_pass(DH, 4, 32)
_edge_pass_o = _make_edge_pass(DO, 8, NCHUNK2, finale=True)


def _dis_from_deg(deg_ref):
    deg = deg_ref[0, :, 0:1] + deg_ref[1, :, 0:1]
    rows = lax.broadcasted_iota(jnp.int32, (NP, 1), 0)
    deg = deg + jnp.where(rows < N, 1.0, 0.0)
    return jnp.where(deg > 0, lax.rsqrt(deg), 0.0)


def _tc1_body(x_ref, w_ref, deg_ref, o_ref):
    dis = _dis_from_deg(deg_ref)
    hh = jnp.dot(x_ref[...], w_ref[...],
                 preferred_element_type=jnp.float32) * dis[:N]
    zt = jnp.zeros((NP - N, DH // 2), jnp.float32)
    o_ref[0] = jnp.concatenate([hh[:, :DH // 2], zt], axis=0)
    o_ref[1] = jnp.concatenate([hh[:, DH // 2:], zt], axis=0)


TC2_R = NP // 8


def _tc2_body(acc_ref, hs1_ref, deg_ref, b1_ref, g_ref, be_ref, w_ref, o_ref, disb_ref):
    row0 = pl.program_id(0) * TC2_R
    deg = deg_ref[0, :, 0:1] + deg_ref[1, :, 0:1]
    rows = row0 + lax.broadcasted_iota(jnp.int32, (TC2_R, 1), 0)
    deg = deg + jnp.where(rows < N, 1.0, 0.0)
    dis = jnp.where(deg > 0, lax.rsqrt(deg), 0.0)
    acc = jnp.concatenate([acc_ref[0] + hs1_ref[0],
                           acc_ref[1] + hs1_ref[1]], axis=1)
    t = acc * dis + b1_ref[...]
    t = jnp.maximum(t, 0.0)
    mu = jnp.mean(t, axis=-1, keepdims=True)
    var = jnp.mean((t - mu) ** 2, axis=-1, keepdims=True)
    h = (t - mu) * lax.rsqrt(var + 1e-5) * g_ref[...] + be_ref[...]
    hs2 = jnp.dot(h, w_ref[...], preferred_element_type=jnp.float32) * dis
    o_ref[0] = hs2[:, :DO // 2]
    o_ref[1] = hs2[:, DO // 2:]
    disb_ref[...] = jnp.broadcast_to(dis, (TC2_R, DO // 2))


_tc1 = pl.pallas_call(_tc1_body,
                      out_shape=jax.ShapeDtypeStruct((NC, NP, DH // 2),
                                                     jnp.float32))
_tc2 = pl.pallas_call(
    _tc2_body,
    grid=(NP // TC2_R,),
    in_specs=[
        pl.BlockSpec((NC, TC2_R, DH // 2), lambda i: (0, i, 0)),
        pl.BlockSpec((NC, TC2_R, DH // 2), lambda i: (0, i, 0)),
        pl.BlockSpec((NC, TC2_R, DEGW), lambda i: (0, i, 0)),
        pl.BlockSpec((1, DH), lambda i: (0, 0)),
        pl.BlockSpec((1, DH), lambda i: (0, 0)),
        pl.BlockSpec((1, DH), lambda i: (0, 0)),
        pl.BlockSpec((DH, DO), lambda i: (0, 0)),
    ],
    out_specs=(pl.BlockSpec((NC, TC2_R, DO // 2), lambda i: (0, i, 0)),
               pl.BlockSpec((TC2_R, DO // 2), lambda i: (i, 0))),
    out_shape=(jax.ShapeDtypeStruct((NC, NP, DO // 2), jnp.float32),
               jax.ShapeDtypeStruct((NP, DO // 2), jnp.float32)))


def kernel(x, edge_index, W1, b1, gamma, beta, W2, b2):
    ei = edge_index.astype(jnp.int32)
    pad = jnp.full((2, EP - E), N, jnp.int32)
    e3 = jnp.concatenate([ei, pad], axis=1)
    e3 = e3.reshape(2, NT * NCHUNK, CH).transpose(1, 0, 2)

    oz = jnp.concatenate([jnp.zeros((64, DEGW), jnp.float32),
                          jnp.ones((CH, DEGW), jnp.float32)])

    degp = _deg_pass(e3, oz)
    hs1 = _tc1(x, W1, degp)
    acc1 = _edge_pass_h(hs1, e3)
    hs2, disb = _tc2(acc1, hs1, degp, b1.reshape(1, DH),
                     gamma.reshape(1, DH), beta.reshape(1, DH), W2)
    out = _edge_pass_o(hs2, e3, disb, b2.reshape(NC, 1, DO // 2))
    return jnp.concatenate([out[0], out[1]], axis=1)[:N]
